# baseline (speedup 1.0000x reference)
import functools

import jax
import jax.numpy as jnp
from jax.experimental import pallas as pl
from jax.experimental.pallas import tpu as pltpu

B, S, D = 1, 2048, 768
E, K, FF = 8, 2, 3072
T = B * S
BLK = 256
NBLK = (T * K) // BLK + E
NPAD = NBLK * BLK


def _router_kernel(x_ref, wlin_ref, blin_ref, wg_ref,
                   lin_ref, topi_ref, topg_ref, xsum_ref):
    xb = x_ref[...]
    lin = jax.lax.dot(xb.astype(jnp.bfloat16),
                      wlin_ref[...].astype(jnp.bfloat16),
                      preferred_element_type=jnp.float32)
    lin = lin + blin_ref[...]
    lin_ref[...] = lin
    xsum_ref[...] = jnp.sum(xb, axis=0, keepdims=True)
    logits = jax.lax.dot(lin, wg_ref[...],
                         preferred_element_type=jnp.float32)
    m = jnp.max(logits, axis=1, keepdims=True)
    p = jnp.exp(logits - m)
    p = p / jnp.sum(p, axis=1, keepdims=True)
    eids = jax.lax.broadcasted_iota(jnp.int32, (T, E), 1)
    v1 = jnp.max(p, axis=1, keepdims=True)
    i1 = jnp.min(jnp.where(p == v1, eids, E), axis=1, keepdims=True)
    p2 = jnp.where(eids == i1, -1.0, p)
    v2 = jnp.max(p2, axis=1, keepdims=True)
    i2 = jnp.min(jnp.where(p2 == v2, eids, E), axis=1, keepdims=True)
    den = v1 + v2
    topi_ref[...] = jnp.concatenate([i1, i2], axis=1)
    topg_ref[...] = jnp.concatenate([v1 / den, v2 / den], axis=1)


def _schedule(topi, topg):
    ef = topi.reshape(T * K)
    pid = jnp.arange(T * K, dtype=jnp.int32)
    counts = jnp.sum(ef[None, :] == jnp.arange(E, dtype=jnp.int32)[:, None],
                     axis=1).astype(jnp.int32)
    nb = (counts + BLK - 1) // BLK
    cnb = jnp.concatenate([jnp.zeros((1,), jnp.int32), jnp.cumsum(nb)])
    ccnt = jnp.concatenate([jnp.zeros((1,), jnp.int32), jnp.cumsum(counts)])
    order = jnp.argsort(ef, stable=True).astype(jnp.int32)
    es = ef[order]
    rank = pid - ccnt[es]
    dest = cnb[es] * BLK + rank
    tok_sched = jnp.zeros((NPAD,), jnp.int32).at[dest].set(order // K)
    gate_sched = jnp.zeros((NPAD,), jnp.float32).at[dest].set(
        topg.reshape(T * K)[order])
    bid = jnp.arange(NBLK, dtype=jnp.int32)
    be = jnp.sum(bid[:, None] >= cnb[None, 1:E], axis=1).astype(jnp.int32)
    total = cnb[E]
    be_last = be[jnp.maximum(total - 1, 0)]
    be = jnp.where(bid < total, be, be_last)
    return (tok_sched.reshape(NBLK, BLK, 1),
            gate_sched.reshape(NBLK, 1, BLK), be)


def _ffn_kernel(be_ref, lin_ref, tok_ref, gate_ref, w1_ref, b1_ref,
                w2_ref, b2_ref, xsum_ref, y_ref, out_ref,
                lin_bf, acc_scr):
    i = pl.program_id(0)

    @pl.when(i == 0)
    def _prologue():
        lin_bf[...] = lin_ref[...].astype(jnp.bfloat16)
        acc_scr[...] = jnp.zeros_like(acc_scr)

    gates = gate_ref[0]
    has_rows = jnp.sum(gates) > 0.0

    @pl.when(has_rows)
    def _block():
        tok = tok_ref[0]
        onehot = (tok == jax.lax.broadcasted_iota(jnp.int32, (BLK, T), 1)
                  ).astype(jnp.bfloat16)
        xg = jax.lax.dot(onehot, lin_bf[...],
                         preferred_element_type=jnp.float32)
        h = jax.lax.dot(xg.astype(jnp.bfloat16),
                        w1_ref[0].astype(jnp.bfloat16),
                        preferred_element_type=jnp.float32)
        h = jax.nn.gelu(h + b1_ref[0])
        eo = jax.lax.dot(h.astype(jnp.bfloat16),
                         w2_ref[0].astype(jnp.bfloat16),
                         preferred_element_type=jnp.float32)
        contrib = jax.lax.dot(gates, eo,
                              preferred_element_type=jnp.float32)
        acc_scr[...] += contrib + jnp.sum(gates) * b2_ref[0]

    @pl.when(i == NBLK - 1)
    def _epilogue():
        sent = (xsum_ref[...] + acc_scr[...]) * (1.0 / T)
        mx = jnp.max(sent)
        lse = mx + jnp.log(jnp.sum(jnp.exp(sent - mx)))
        cls = jax.lax.broadcasted_iota(jnp.int32, (1, D), 1)
        picked = jnp.sum(jnp.where(cls == y_ref[0, 0], sent, 0.0))
        out_ref[...] = jnp.broadcast_to(lse - picked, (1, 1))


@jax.jit
def _run(x, y, W_lin, b_lin, Wg, W1, b1, W2, b2):
    x2 = x.reshape(T, D)
    y32 = y.astype(jnp.int32).reshape(1, 1)

    lin, topi, topg, xsum = pl.pallas_call(
        _router_kernel,
        in_specs=[pl.BlockSpec((T, D), lambda: (0, 0)),
                  pl.BlockSpec((D, D), lambda: (0, 0)),
                  pl.BlockSpec((1, D), lambda: (0, 0)),
                  pl.BlockSpec((D, E), lambda: (0, 0))],
        out_specs=[pl.BlockSpec((T, D), lambda: (0, 0)),
                   pl.BlockSpec((T, K), lambda: (0, 0)),
                   pl.BlockSpec((T, K), lambda: (0, 0)),
                   pl.BlockSpec((1, D), lambda: (0, 0))],
        out_shape=[jax.ShapeDtypeStruct((T, D), jnp.float32),
                   jax.ShapeDtypeStruct((T, K), jnp.int32),
                   jax.ShapeDtypeStruct((T, K), jnp.float32),
                   jax.ShapeDtypeStruct((1, D), jnp.float32)],
    )(x2, W_lin, b_lin.reshape(1, D), Wg)

    tok_sched, gate_sched, block_expert = _schedule(topi, topg)

    grid_spec = pltpu.PrefetchScalarGridSpec(
        num_scalar_prefetch=1,
        grid=(NBLK,),
        in_specs=[
            pl.BlockSpec((T, D), lambda i, be: (0, 0)),
            pl.BlockSpec((1, BLK, 1), lambda i, be: (i, 0, 0)),
            pl.BlockSpec((1, 1, BLK), lambda i, be: (i, 0, 0)),
            pl.BlockSpec((1, D, FF), lambda i, be: (be[i], 0, 0)),
            pl.BlockSpec((1, 1, FF), lambda i, be: (be[i], 0, 0)),
            pl.BlockSpec((1, FF, D), lambda i, be: (be[i], 0, 0)),
            pl.BlockSpec((1, 1, D), lambda i, be: (be[i], 0, 0)),
            pl.BlockSpec((1, D), lambda i, be: (0, 0)),
            pl.BlockSpec(memory_space=pltpu.SMEM),
        ],
        out_specs=pl.BlockSpec((1, 1), lambda i, be: (0, 0)),
        scratch_shapes=[pltpu.VMEM((T, D), jnp.bfloat16),
                        pltpu.VMEM((1, D), jnp.float32)],
    )
    out = pl.pallas_call(
        _ffn_kernel,
        grid_spec=grid_spec,
        out_shape=jax.ShapeDtypeStruct((1, 1), jnp.float32),
        compiler_params=pltpu.CompilerParams(
            dimension_semantics=("arbitrary",),
        ),
    )(block_expert, lin, tok_sched, gate_sched, W1, b1.reshape(E, 1, FF),
      W2, b2.reshape(E, 1, D), xsum, y32)
    return out[0, 0]


def kernel(x, y, W_lin, b_lin, Wg, W1, b1, W2, b2):
    return _run(x, y, W_lin, b_lin, Wg, W1, b1, W2, b2)


# speedup vs baseline: 1.0417x; 1.0417x over previous
import dataclasses
import functools

import jax
import jax.numpy as jnp
from jax import lax
from jax.experimental import pallas as pl
from jax.experimental.pallas import tpu as pltpu
from jax.experimental.pallas import tpu_sc as plsc

B, S, D = 1, 2048, 768
E, K, FF = 8, 2, 3072
T = B * S
BLK = 256
NBLK = (T * K) // BLK + E
NPAD = NBLK * BLK

NT = 16
CHUNK = (T * K) // NT
ZF = NPAD // NT


def _router_kernel(x_ref, wlin_ref, blin_ref, wg_ref,
                   lin_ref, topi_ref, topg_ref, xsum_ref):
    xb = x_ref[...]
    lin = jax.lax.dot(xb.astype(jnp.bfloat16),
                      wlin_ref[...].astype(jnp.bfloat16),
                      preferred_element_type=jnp.float32)
    lin = lin + blin_ref[...]
    lin_ref[...] = lin
    xsum_ref[...] = jnp.sum(xb, axis=0, keepdims=True)
    logits = jax.lax.dot(lin, wg_ref[...],
                         preferred_element_type=jnp.float32)
    m = jnp.max(logits, axis=1, keepdims=True)
    p = jnp.exp(logits - m)
    p = p / jnp.sum(p, axis=1, keepdims=True)
    eids = jax.lax.broadcasted_iota(jnp.int32, (T, E), 1)
    v1 = jnp.max(p, axis=1, keepdims=True)
    i1 = jnp.min(jnp.where(p == v1, eids, E), axis=1, keepdims=True)
    p2 = jnp.where(eids == i1, -1.0, p)
    v2 = jnp.max(p2, axis=1, keepdims=True)
    i2 = jnp.min(jnp.where(p2 == v2, eids, E), axis=1, keepdims=True)
    den = v1 + v2
    topi_ref[...] = jnp.concatenate([i1, i2], axis=1)
    topg_ref[...] = jnp.concatenate([v1 / den, v2 / den], axis=1)


def _dispatch_body(topi_hbm, topg_hbm, tok_hbm, gate_hbm, be_hbm,
                   e_vmem, g_vmem, tok_vmem, dest_vmem, cnt_vmem,
                   zf_i, zf_f, be_vmem, counts_vmem, counts_sh):
    wid = lax.axis_index("s")
    cid = lax.axis_index("c")
    lanes = jnp.arange(16, dtype=jnp.int32)
    zero16 = jnp.zeros((16,), jnp.int32)
    on_core0 = cid == 0

    def bc(s):
        return jnp.broadcast_to(s, (16,))

    @pl.when(on_core0)
    def _phase1():
        base = wid * CHUNK
        pltpu.sync_copy(topi_hbm.at[pl.ds(base, CHUNK)], e_vmem)
        for h in range(2):
            pltpu.sync_copy(topg_hbm.at[pl.ds(base + 128 * h, 128)],
                            g_vmem.at[h])
        cnt = [jnp.int32(0)] * E
        for k in range(CHUNK // 16):
            v = e_vmem[pl.ds(16 * k, 16)]
            for e in range(E):
                cnt[e] = cnt[e] + jnp.sum((v == e).astype(jnp.int32))
        cvec = jnp.zeros((16,), jnp.int32)
        for e in range(E):
            cvec = jnp.where(lanes == e, bc(cnt[e]), cvec)
        cnt_vmem[...] = cvec
        zi = jnp.zeros((16,), jnp.int32)
        zf32 = jnp.zeros((16,), jnp.float32)
        for j in range(ZF // 16):
            zf_i[pl.ds(16 * j, 16)] = zi
            zf_f[pl.ds(16 * j, 16)] = zf32
        pltpu.sync_copy(zf_i, tok_hbm.at[pl.ds(wid * ZF, ZF)])
        pltpu.sync_copy(zf_f, gate_hbm.at[pl.ds(wid * ZF, ZF)])
        pltpu.sync_copy(cnt_vmem.at[pl.ds(0, 8)],
                        counts_sh.at[pl.ds(wid * 8, 8)])

    plsc.subcore_barrier()

    @pl.when(on_core0)
    def _phase2():
        base = wid * CHUNK
        pltpu.sync_copy(counts_sh, counts_vmem)
        tot16 = jnp.zeros((16,), jnp.int32)
        pfx16 = jnp.zeros((16,), jnp.int32)
        for k in range(NT // 2):
            ck = counts_vmem[pl.ds(16 * k, 16)]
            tot16 = tot16 + ck
            wvec = 2 * k + (lanes >= 8).astype(jnp.int32)
            pfx16 = pfx16 + jnp.where(wvec < bc(wid), ck, zero16)
        lane_e = lanes & 7
        tot = [jnp.sum(jnp.where(lane_e == e, tot16, zero16))
               for e in range(E)]
        pfx = [jnp.sum(jnp.where(lane_e == e, pfx16, zero16))
               for e in range(E)]
        nb = [(tot[e] + (BLK - 1)) // BLK for e in range(E)]
        cnb = [jnp.int32(0)] * E
        for e in range(1, E):
            cnb[e] = cnb[e - 1] + nb[e - 1]
        basee = [cnb[e] * BLK + pfx[e] for e in range(E)]
        carry = [jnp.int32(0)] * E
        for k in range(CHUNK // 16):
            v = e_vmem[pl.ds(16 * k, 16)]
            dk = jnp.zeros((16,), jnp.int32)
            for e in range(E):
                mask = v == e
                mi = mask.astype(jnp.int32)
                rank = plsc.cumsum(mi) - 1
                dk = jnp.where(mask, bc(basee[e] + carry[e]) + rank, dk)
                carry[e] = carry[e] + jnp.sum(mi)
            h, s = k // 8, k % 8
            dest_vmem[h, pl.ds(16 * s, 16)] = dk
            pairp = bc(base + 16 * k) + lanes
            tok_vmem[h, pl.ds(16 * s, 16)] = pairp // K
        for h in range(2):
            pltpu.sync_copy(tok_vmem.at[h], tok_hbm.at[dest_vmem.at[h]])
            pltpu.sync_copy(g_vmem.at[h], gate_hbm.at[dest_vmem.at[h]])

        @pl.when(wid == 0)
        def _bmap():
            total = cnb[E - 1] + nb[E - 1]
            belast = jnp.int32(0)
            for e in range(1, E):
                belast = belast + (total - 1 >= cnb[e]).astype(jnp.int32)
            for j in range(2):
                bvec = lanes + 16 * j
                acc = jnp.zeros((16,), jnp.int32)
                for e in range(1, E):
                    acc = acc + (bvec >= bc(cnb[e])).astype(jnp.int32)
                acc = jnp.where(bvec < bc(total), acc, bc(belast))
                be_vmem[pl.ds(16 * j, 16)] = acc
            pltpu.sync_copy(be_vmem.at[pl.ds(0, NBLK)], be_hbm)


def _sc_dispatch(topi, topg):
    mesh = plsc.VectorSubcoreMesh(core_axis_name="c", subcore_axis_name="s")
    cp = pltpu.CompilerParams()
    if "needs_layout_passes" in pltpu.CompilerParams.__dataclass_fields__:
        cp = dataclasses.replace(cp, needs_layout_passes=False)
    fn = pl.kernel(
        _dispatch_body, mesh=mesh, compiler_params=cp,
        out_type=[jax.ShapeDtypeStruct((NPAD,), jnp.int32),
                  jax.ShapeDtypeStruct((NPAD,), jnp.float32),
                  jax.ShapeDtypeStruct((NBLK,), jnp.int32)],
        scratch_types=[pltpu.VMEM((CHUNK,), jnp.int32),
                       pltpu.VMEM((2, 128), jnp.float32),
                       pltpu.VMEM((2, 128), jnp.int32),
                       pltpu.VMEM((2, 128), jnp.int32),
                       pltpu.VMEM((16,), jnp.int32),
                       pltpu.VMEM((ZF,), jnp.int32),
                       pltpu.VMEM((ZF,), jnp.float32),
                       pltpu.VMEM((32,), jnp.int32),
                       pltpu.VMEM((NT * 8,), jnp.int32),
                       pltpu.VMEM_SHARED((NT * 8,), jnp.int32)],
    )
    tok_flat, gate_flat, block_expert = fn(topi.reshape(T * K),
                                           topg.reshape(T * K))
    return (tok_flat.reshape(NBLK, BLK, 1),
            gate_flat.reshape(NBLK, 1, BLK), block_expert)


def _ffn_kernel(be_ref, lin_ref, tok_ref, gate_ref, w1_ref, b1_ref,
                w2_ref, b2_ref, xsum_ref, y_ref, out_ref,
                lin_bf, acc_scr):
    i = pl.program_id(0)

    @pl.when(i == 0)
    def _prologue():
        lin_bf[...] = lin_ref[...].astype(jnp.bfloat16)
        acc_scr[...] = jnp.zeros_like(acc_scr)

    gates = gate_ref[0]
    has_rows = jnp.sum(gates) > 0.0

    @pl.when(has_rows)
    def _block():
        tok = tok_ref[0]
        onehot = (tok == jax.lax.broadcasted_iota(jnp.int32, (BLK, T), 1)
                  ).astype(jnp.bfloat16)
        xg = jax.lax.dot(onehot, lin_bf[...],
                         preferred_element_type=jnp.float32)
        h = jax.lax.dot(xg.astype(jnp.bfloat16),
                        w1_ref[0].astype(jnp.bfloat16),
                        preferred_element_type=jnp.float32)
        h = jax.nn.gelu(h + b1_ref[0])
        eo = jax.lax.dot(h.astype(jnp.bfloat16),
                         w2_ref[0].astype(jnp.bfloat16),
                         preferred_element_type=jnp.float32)
        contrib = jax.lax.dot(gates, eo,
                              preferred_element_type=jnp.float32)
        acc_scr[...] += contrib + jnp.sum(gates) * b2_ref[0]

    @pl.when(i == NBLK - 1)
    def _epilogue():
        sent = (xsum_ref[...] + acc_scr[...]) * (1.0 / T)
        mx = jnp.max(sent)
        lse = mx + jnp.log(jnp.sum(jnp.exp(sent - mx)))
        cls = jax.lax.broadcasted_iota(jnp.int32, (1, D), 1)
        picked = jnp.sum(jnp.where(cls == y_ref[0, 0], sent, 0.0))
        out_ref[...] = jnp.broadcast_to(lse - picked, (1, 1))


@jax.jit
def _run(x, y, W_lin, b_lin, Wg, W1, b1, W2, b2):
    x2 = x.reshape(T, D)
    y32 = y.astype(jnp.int32).reshape(1, 1)

    lin, topi, topg, xsum = pl.pallas_call(
        _router_kernel,
        in_specs=[pl.BlockSpec((T, D), lambda: (0, 0)),
                  pl.BlockSpec((D, D), lambda: (0, 0)),
                  pl.BlockSpec((1, D), lambda: (0, 0)),
                  pl.BlockSpec((D, E), lambda: (0, 0))],
        out_specs=[pl.BlockSpec((T, D), lambda: (0, 0)),
                   pl.BlockSpec((T, K), lambda: (0, 0)),
                   pl.BlockSpec((T, K), lambda: (0, 0)),
                   pl.BlockSpec((1, D), lambda: (0, 0))],
        out_shape=[jax.ShapeDtypeStruct((T, D), jnp.float32),
                   jax.ShapeDtypeStruct((T, K), jnp.int32),
                   jax.ShapeDtypeStruct((T, K), jnp.float32),
                   jax.ShapeDtypeStruct((1, D), jnp.float32)],
    )(x2, W_lin, b_lin.reshape(1, D), Wg)

    tok_sched, gate_sched, block_expert = _sc_dispatch(topi, topg)

    grid_spec = pltpu.PrefetchScalarGridSpec(
        num_scalar_prefetch=1,
        grid=(NBLK,),
        in_specs=[
            pl.BlockSpec((T, D), lambda i, be: (0, 0)),
            pl.BlockSpec((1, BLK, 1), lambda i, be: (i, 0, 0)),
            pl.BlockSpec((1, 1, BLK), lambda i, be: (i, 0, 0)),
            pl.BlockSpec((1, D, FF), lambda i, be: (be[i], 0, 0)),
            pl.BlockSpec((1, 1, FF), lambda i, be: (be[i], 0, 0)),
            pl.BlockSpec((1, FF, D), lambda i, be: (be[i], 0, 0)),
            pl.BlockSpec((1, 1, D), lambda i, be: (be[i], 0, 0)),
            pl.BlockSpec((1, D), lambda i, be: (0, 0)),
            pl.BlockSpec(memory_space=pltpu.SMEM),
        ],
        out_specs=pl.BlockSpec((1, 1), lambda i, be: (0, 0)),
        scratch_shapes=[pltpu.VMEM((T, D), jnp.bfloat16),
                        pltpu.VMEM((1, D), jnp.float32)],
    )
    out = pl.pallas_call(
        _ffn_kernel,
        grid_spec=grid_spec,
        out_shape=jax.ShapeDtypeStruct((1, 1), jnp.float32),
        compiler_params=pltpu.CompilerParams(
            dimension_semantics=("arbitrary",),
        ),
    )(block_expert, lin, tok_sched, gate_sched, W1, b1.reshape(E, 1, FF),
      W2, b2.reshape(E, 1, D), xsum, y32)
    return out[0, 0]


def kernel(x, y, W_lin, b_lin, Wg, W1, b1, W2, b2):
    return _run(x, y, W_lin, b_lin, Wg, W1, b1, W2, b2)


# speedup vs baseline: 1.1416x; 1.0959x over previous
import dataclasses
import functools

import jax
import jax.numpy as jnp
from jax import lax
from jax.experimental import pallas as pl
from jax.experimental.pallas import tpu as pltpu
from jax.experimental.pallas import tpu_sc as plsc

B, S, D = 1, 2048, 768
E, K, FF = 8, 2, 3072
T = B * S
BLK = 256
NBLK = (T * K) // BLK + E
NPAD = NBLK * BLK

NT = 16
CHUNK = (T * K) // NT
ZF = NPAD // NT


def _router_kernel(x_ref, wlin_ref, blin_ref, wg_ref, topi_ref, topg_ref):
    wc = jax.lax.dot(wlin_ref[...], wg_ref[...],
                     preferred_element_type=jnp.float32)
    logits = (jax.lax.dot(x_ref[...], wc,
                          preferred_element_type=jnp.float32)
              + jax.lax.dot(blin_ref[...], wg_ref[...],
                            preferred_element_type=jnp.float32))
    m = jnp.max(logits, axis=1, keepdims=True)
    p = jnp.exp(logits - m)
    p = p / jnp.sum(p, axis=1, keepdims=True)
    eids = jax.lax.broadcasted_iota(jnp.int32, (T, E), 1)
    v1 = jnp.max(p, axis=1, keepdims=True)
    i1 = jnp.min(jnp.where(p == v1, eids, E), axis=1, keepdims=True)
    p2 = jnp.where(eids == i1, -1.0, p)
    v2 = jnp.max(p2, axis=1, keepdims=True)
    i2 = jnp.min(jnp.where(p2 == v2, eids, E), axis=1, keepdims=True)
    den = v1 + v2
    topi_ref[...] = jnp.concatenate([i1, i2], axis=1)
    topg_ref[...] = jnp.concatenate([v1 / den, v2 / den], axis=1)


def _lin_kernel(x_ref, wlin_ref, blin_ref, lin_ref, xsum_ref):
    xb = x_ref[...]
    lin = jax.lax.dot(xb.astype(jnp.bfloat16),
                      wlin_ref[...].astype(jnp.bfloat16),
                      preferred_element_type=jnp.float32)
    lin_ref[...] = (lin + blin_ref[...]).astype(jnp.bfloat16)
    xsum_ref[...] = jnp.sum(xb, axis=0, keepdims=True)


def _dispatch_body(topi_hbm, topg_hbm, tok_hbm, gate_hbm, be_hbm,
                   e_vmem, g_vmem, tok_vmem, dest_vmem, cnt_vmem,
                   sc0_vmem, sc1_vmem, zf_i, zf_f, be_vmem,
                   counts_vmem, counts_sh, sem0, sem1, sem2, sem3):
    wid = lax.axis_index("s")
    cid = lax.axis_index("c")
    lanes = jnp.arange(16, dtype=jnp.int32)
    zero16 = jnp.zeros((16,), jnp.int32)
    on_core0 = cid == 0

    def splat(x, e):
        return plsc.load_gather(x, [jnp.full((16,), e, jnp.int32)])

    @pl.when(on_core0)
    def _phase1():
        base = wid * CHUNK
        cp_e = pltpu.make_async_copy(topi_hbm.at[pl.ds(base, CHUNK)],
                                     e_vmem, sem0)
        cp_e.start()
        cp_g0 = pltpu.make_async_copy(topg_hbm.at[pl.ds(base, 128)],
                                      g_vmem.at[0], sem1)
        cp_g0.start()
        cp_g1 = pltpu.make_async_copy(topg_hbm.at[pl.ds(base + 128, 128)],
                                      g_vmem.at[1], sem2)
        cp_g1.start()
        zi = jnp.zeros((16,), jnp.int32)
        zf32 = jnp.zeros((16,), jnp.float32)
        for j in range(ZF // 16):
            zf_i[pl.ds(16 * j, 16)] = zi
            zf_f[pl.ds(16 * j, 16)] = zf32
        cp_z0 = pltpu.make_async_copy(zf_i, tok_hbm.at[pl.ds(wid * ZF, ZF)],
                                      sem3)
        cp_z0.start()
        cp_e.wait()
        cnt = [zero16] * E
        for k in range(CHUNK // 16):
            v = e_vmem[pl.ds(16 * k, 16)]
            for e in range(E):
                cnt[e] = cnt[e] + plsc.all_reduce_population_count(v == e)
        cvec = zero16
        for e in range(E):
            cvec = jnp.where(lanes == e, cnt[e], cvec)
        cnt_vmem[...] = cvec
        pltpu.sync_copy(cnt_vmem.at[pl.ds(0, 8)],
                        counts_sh.at[pl.ds(wid * 8, 8)])
        cp_z0.wait()
        cp_z1 = pltpu.make_async_copy(zf_f, gate_hbm.at[pl.ds(wid * ZF, ZF)],
                                      sem3)
        cp_z1.start()
        cp_g0.wait()
        cp_g1.wait()
        cp_z1.wait()

    plsc.subcore_barrier()

    @pl.when(on_core0)
    def _phase2():
        base = wid * CHUNK
        pltpu.sync_copy(counts_sh, counts_vmem)
        tot16 = zero16
        pfx16 = zero16
        for k in range(NT // 2):
            ck = counts_vmem[pl.ds(16 * k, 16)]
            tot16 = tot16 + ck
            wvec = 2 * k + (lanes >= 8).astype(jnp.int32)
            pfx16 = pfx16 + jnp.where(wvec < jnp.broadcast_to(wid, (16,)),
                                      ck, zero16)
        sc0_vmem[...] = tot16
        sc1_vmem[...] = pfx16
        tot = [splat(sc0_vmem, e) + splat(sc0_vmem, e + 8) for e in range(E)]
        pfx = [splat(sc1_vmem, e) + splat(sc1_vmem, e + 8) for e in range(E)]
        nb = [(tot[e] + (BLK - 1)) // BLK for e in range(E)]
        cnb = [zero16] * E
        for e in range(1, E):
            cnb[e] = cnb[e - 1] + nb[e - 1]
        basee = [cnb[e] * BLK + pfx[e] for e in range(E)]
        carry = [zero16] * E
        for k in range(CHUNK // 16):
            v = e_vmem[pl.ds(16 * k, 16)]
            dk = zero16
            for e in range(E):
                mask = v == e
                rank = plsc.cumsum(mask.astype(jnp.int32)) - 1
                dk = jnp.where(mask, basee[e] + carry[e] + rank, dk)
                carry[e] = carry[e] + plsc.all_reduce_population_count(mask)
            h, s = k // 8, k % 8
            dest_vmem[h, pl.ds(16 * s, 16)] = dk
            pairp = jnp.broadcast_to(base + 16 * k, (16,)) + lanes
            tok_vmem[h, pl.ds(16 * s, 16)] = pairp // K
        cps = []
        for h in range(2):
            c1 = pltpu.make_async_copy(tok_vmem.at[h],
                                       tok_hbm.at[dest_vmem.at[h]],
                                       sem0 if h == 0 else sem1)
            c1.start()
            c2 = pltpu.make_async_copy(g_vmem.at[h],
                                       gate_hbm.at[dest_vmem.at[h]],
                                       sem2 if h == 0 else sem3)
            c2.start()
            cps += [c1, c2]
        for c in cps:
            c.wait()

        @pl.when(wid == 0)
        def _bmap():
            total = cnb[E - 1] + nb[E - 1]
            belast = zero16
            for e in range(1, E):
                belast = belast + (total - 1 >= cnb[e]).astype(jnp.int32)
            for j in range(2):
                bvec = lanes + 16 * j
                acc = zero16
                for e in range(1, E):
                    acc = acc + (bvec >= cnb[e]).astype(jnp.int32)
                acc = jnp.where(bvec < total, acc, belast)
                be_vmem[pl.ds(16 * j, 16)] = acc
            pltpu.sync_copy(be_vmem.at[pl.ds(0, NBLK)], be_hbm)


def _sc_dispatch(topi, topg):
    mesh = plsc.VectorSubcoreMesh(core_axis_name="c", subcore_axis_name="s")
    cp = pltpu.CompilerParams()
    if "needs_layout_passes" in pltpu.CompilerParams.__dataclass_fields__:
        cp = dataclasses.replace(cp, needs_layout_passes=False)
    fn = pl.kernel(
        _dispatch_body, mesh=mesh, compiler_params=cp,
        out_type=[jax.ShapeDtypeStruct((NPAD,), jnp.int32),
                  jax.ShapeDtypeStruct((NPAD,), jnp.float32),
                  jax.ShapeDtypeStruct((NBLK,), jnp.int32)],
        scratch_types=[pltpu.VMEM((CHUNK,), jnp.int32),
                       pltpu.VMEM((2, 128), jnp.float32),
                       pltpu.VMEM((2, 128), jnp.int32),
                       pltpu.VMEM((2, 128), jnp.int32),
                       pltpu.VMEM((16,), jnp.int32),
                       pltpu.VMEM((16,), jnp.int32),
                       pltpu.VMEM((16,), jnp.int32),
                       pltpu.VMEM((ZF,), jnp.int32),
                       pltpu.VMEM((ZF,), jnp.float32),
                       pltpu.VMEM((32,), jnp.int32),
                       pltpu.VMEM((NT * 8,), jnp.int32),
                       pltpu.VMEM_SHARED((NT * 8,), jnp.int32),
                       pltpu.SemaphoreType.DMA,
                       pltpu.SemaphoreType.DMA,
                       pltpu.SemaphoreType.DMA,
                       pltpu.SemaphoreType.DMA],
    )
    tok_flat, gate_flat, block_expert = fn(topi.reshape(T * K),
                                           topg.reshape(T * K))
    return (tok_flat.reshape(NBLK, BLK, 1),
            gate_flat.reshape(NBLK, 1, BLK), block_expert)


def _ffn_kernel(be_ref, lin_ref, tok_ref, gate_ref, w1_ref, b1_ref,
                w2_ref, b2_ref, xsum_ref, y_ref, out_ref, acc_scr):
    i = pl.program_id(0)

    @pl.when(i == 0)
    def _prologue():
        acc_scr[...] = jnp.zeros_like(acc_scr)

    gates = gate_ref[0]
    has_rows = jnp.sum(gates) > 0.0

    @pl.when(has_rows)
    def _block():
        tok = tok_ref[0]
        onehot = (tok == jax.lax.broadcasted_iota(jnp.int32, (BLK, T), 1)
                  ).astype(jnp.bfloat16)
        xg = jax.lax.dot(onehot, lin_ref[...],
                         preferred_element_type=jnp.float32)
        h = jax.lax.dot(xg.astype(jnp.bfloat16),
                        w1_ref[0].astype(jnp.bfloat16),
                        preferred_element_type=jnp.float32)
        h = jax.nn.gelu(h + b1_ref[0])
        eo = jax.lax.dot(h.astype(jnp.bfloat16),
                         w2_ref[0].astype(jnp.bfloat16),
                         preferred_element_type=jnp.float32)
        contrib = jax.lax.dot(gates, eo,
                              preferred_element_type=jnp.float32)
        acc_scr[...] += contrib + jnp.sum(gates) * b2_ref[0]

    @pl.when(i == NBLK - 1)
    def _epilogue():
        sent = (xsum_ref[...] + acc_scr[...]) * (1.0 / T)
        mx = jnp.max(sent)
        lse = mx + jnp.log(jnp.sum(jnp.exp(sent - mx)))
        cls = jax.lax.broadcasted_iota(jnp.int32, (1, D), 1)
        picked = jnp.sum(jnp.where(cls == y_ref[0, 0], sent, 0.0))
        out_ref[...] = jnp.broadcast_to(lse - picked, (1, 1))


@jax.jit
def _run(x, y, W_lin, b_lin, Wg, W1, b1, W2, b2):
    x2 = x.reshape(T, D)
    y32 = y.astype(jnp.int32).reshape(1, 1)
    blin2 = b_lin.reshape(1, D)

    topi, topg = pl.pallas_call(
        _router_kernel,
        in_specs=[pl.BlockSpec((T, D), lambda: (0, 0)),
                  pl.BlockSpec((D, D), lambda: (0, 0)),
                  pl.BlockSpec((1, D), lambda: (0, 0)),
                  pl.BlockSpec((D, E), lambda: (0, 0))],
        out_specs=[pl.BlockSpec((T, K), lambda: (0, 0)),
                   pl.BlockSpec((T, K), lambda: (0, 0))],
        out_shape=[jax.ShapeDtypeStruct((T, K), jnp.int32),
                   jax.ShapeDtypeStruct((T, K), jnp.float32)],
    )(x2, W_lin, blin2, Wg)

    lin, xsum = pl.pallas_call(
        _lin_kernel,
        in_specs=[pl.BlockSpec((T, D), lambda: (0, 0)),
                  pl.BlockSpec((D, D), lambda: (0, 0)),
                  pl.BlockSpec((1, D), lambda: (0, 0))],
        out_specs=[pl.BlockSpec((T, D), lambda: (0, 0)),
                   pl.BlockSpec((1, D), lambda: (0, 0))],
        out_shape=[jax.ShapeDtypeStruct((T, D), jnp.bfloat16),
                   jax.ShapeDtypeStruct((1, D), jnp.float32)],
    )(x2, W_lin, blin2)

    tok_sched, gate_sched, block_expert = _sc_dispatch(topi, topg)

    grid_spec = pltpu.PrefetchScalarGridSpec(
        num_scalar_prefetch=1,
        grid=(NBLK,),
        in_specs=[
            pl.BlockSpec((T, D), lambda i, be: (0, 0)),
            pl.BlockSpec((1, BLK, 1), lambda i, be: (i, 0, 0)),
            pl.BlockSpec((1, 1, BLK), lambda i, be: (i, 0, 0)),
            pl.BlockSpec((1, D, FF), lambda i, be: (be[i], 0, 0)),
            pl.BlockSpec((1, 1, FF), lambda i, be: (be[i], 0, 0)),
            pl.BlockSpec((1, FF, D), lambda i, be: (be[i], 0, 0)),
            pl.BlockSpec((1, 1, D), lambda i, be: (be[i], 0, 0)),
            pl.BlockSpec((1, D), lambda i, be: (0, 0)),
            pl.BlockSpec(memory_space=pltpu.SMEM),
        ],
        out_specs=pl.BlockSpec((1, 1), lambda i, be: (0, 0)),
        scratch_shapes=[pltpu.VMEM((1, D), jnp.float32)],
    )
    out = pl.pallas_call(
        _ffn_kernel,
        grid_spec=grid_spec,
        out_shape=jax.ShapeDtypeStruct((1, 1), jnp.float32),
        compiler_params=pltpu.CompilerParams(
            dimension_semantics=("arbitrary",),
        ),
    )(block_expert, lin, tok_sched, gate_sched, W1, b1.reshape(E, 1, FF),
      W2, b2.reshape(E, 1, D), xsum, y32)
    return out[0, 0]


def kernel(x, y, W_lin, b_lin, Wg, W1, b1, W2, b2):
    return _run(x, y, W_lin, b_lin, Wg, W1, b1, W2, b2)


# speedup vs baseline: 1.2567x; 1.1009x over previous
import dataclasses
import functools

import jax
import jax.numpy as jnp
from jax import lax
from jax.experimental import pallas as pl
from jax.experimental.pallas import tpu as pltpu
from jax.experimental.pallas import tpu_sc as plsc

B, S, D = 1, 2048, 768
E, K, FF = 8, 2, 3072
T = B * S
BLK = 256
NBLK = (T * K) // BLK + E
NPAD = NBLK * BLK

NT = 16
CHUNK = (T * K) // NT
ZF = NPAD // NT


def _router_kernel(x_ref, wlin_ref, blin_ref, wg_ref, topi_ref, topg_ref):
    wc = jax.lax.dot(wlin_ref[...], wg_ref[...],
                     preferred_element_type=jnp.float32)
    logits = (jax.lax.dot(x_ref[...], wc,
                          preferred_element_type=jnp.float32)
              + jax.lax.dot(blin_ref[...], wg_ref[...],
                            preferred_element_type=jnp.float32))
    m = jnp.max(logits, axis=1, keepdims=True)
    p = jnp.exp(logits - m)
    p = p / jnp.sum(p, axis=1, keepdims=True)
    eids = jax.lax.broadcasted_iota(jnp.int32, (T, E), 1)
    v1 = jnp.max(p, axis=1, keepdims=True)
    i1 = jnp.min(jnp.where(p == v1, eids, E), axis=1, keepdims=True)
    p2 = jnp.where(eids == i1, -1.0, p)
    v2 = jnp.max(p2, axis=1, keepdims=True)
    i2 = jnp.min(jnp.where(p2 == v2, eids, E), axis=1, keepdims=True)
    den = v1 + v2
    topi_ref[...] = jnp.concatenate([i1, i2], axis=1)
    topg_ref[...] = jnp.concatenate([v1 / den, v2 / den], axis=1)


def _lin_kernel(x_ref, wlin_ref, blin_ref, lin_ref, xsum_ref):
    xb = x_ref[...]
    lin = jax.lax.dot(xb.astype(jnp.bfloat16),
                      wlin_ref[...].astype(jnp.bfloat16),
                      preferred_element_type=jnp.float32)
    lin_ref[...] = (lin + blin_ref[...]).astype(jnp.float8_e4m3fn)
    xsum_ref[...] = jnp.sum(xb, axis=0, keepdims=True)


def _dispatch_body(topi_hbm, topg_hbm, tok_hbm, gate_hbm, be_hbm,
                   e_vmem, g_vmem, tok_vmem, dest_vmem, cnt_vmem,
                   sc0_vmem, sc1_vmem, zf_i, zf_f, be_vmem,
                   counts_vmem, counts_sh, sem0, sem1, sem2, sem3):
    wid = lax.axis_index("s")
    cid = lax.axis_index("c")
    lanes = jnp.arange(16, dtype=jnp.int32)
    zero16 = jnp.zeros((16,), jnp.int32)
    on_core0 = cid == 0

    def splat(x, e):
        return plsc.load_gather(x, [jnp.full((16,), e, jnp.int32)])

    @pl.when(on_core0)
    def _phase1():
        base = wid * CHUNK
        cp_e = pltpu.make_async_copy(topi_hbm.at[pl.ds(base, CHUNK)],
                                     e_vmem, sem0)
        cp_e.start()
        cp_g0 = pltpu.make_async_copy(topg_hbm.at[pl.ds(base, 128)],
                                      g_vmem.at[0], sem1)
        cp_g0.start()
        cp_g1 = pltpu.make_async_copy(topg_hbm.at[pl.ds(base + 128, 128)],
                                      g_vmem.at[1], sem2)
        cp_g1.start()
        zi = jnp.zeros((16,), jnp.int32)
        zf32 = jnp.zeros((16,), jnp.float32)
        for j in range(ZF // 16):
            zf_i[pl.ds(16 * j, 16)] = zi
            zf_f[pl.ds(16 * j, 16)] = zf32
        cp_z0 = pltpu.make_async_copy(zf_i, tok_hbm.at[pl.ds(wid * ZF, ZF)],
                                      sem3)
        cp_z0.start()
        cp_e.wait()
        cnt = [zero16] * E
        for k in range(CHUNK // 16):
            v = e_vmem[pl.ds(16 * k, 16)]
            for e in range(E):
                cnt[e] = cnt[e] + plsc.all_reduce_population_count(v == e)
        cvec = zero16
        for e in range(E):
            cvec = jnp.where(lanes == e, cnt[e], cvec)
        cnt_vmem[...] = cvec
        pltpu.sync_copy(cnt_vmem.at[pl.ds(0, 8)],
                        counts_sh.at[pl.ds(wid * 8, 8)])
        cp_z0.wait()
        cp_z1 = pltpu.make_async_copy(zf_f, gate_hbm.at[pl.ds(wid * ZF, ZF)],
                                      sem3)
        cp_z1.start()
        cp_g0.wait()
        cp_g1.wait()
        cp_z1.wait()

    plsc.subcore_barrier()

    @pl.when(on_core0)
    def _phase2():
        base = wid * CHUNK
        pltpu.sync_copy(counts_sh, counts_vmem)
        tot16 = zero16
        pfx16 = zero16
        for k in range(NT // 2):
            ck = counts_vmem[pl.ds(16 * k, 16)]
            tot16 = tot16 + ck
            wvec = 2 * k + (lanes >= 8).astype(jnp.int32)
            pfx16 = pfx16 + jnp.where(wvec < jnp.broadcast_to(wid, (16,)),
                                      ck, zero16)
        sc0_vmem[...] = tot16
        sc1_vmem[...] = pfx16
        tot = [splat(sc0_vmem, e) + splat(sc0_vmem, e + 8) for e in range(E)]
        pfx = [splat(sc1_vmem, e) + splat(sc1_vmem, e + 8) for e in range(E)]
        nb = [(tot[e] + (BLK - 1)) // BLK for e in range(E)]
        cnb = [zero16] * E
        for e in range(1, E):
            cnb[e] = cnb[e - 1] + nb[e - 1]
        basee = [cnb[e] * BLK + pfx[e] for e in range(E)]
        carry = [zero16] * E
        for k in range(CHUNK // 16):
            v = e_vmem[pl.ds(16 * k, 16)]
            dk = zero16
            for e in range(E):
                mask = v == e
                rank = plsc.cumsum(mask.astype(jnp.int32)) - 1
                dk = jnp.where(mask, basee[e] + carry[e] + rank, dk)
                carry[e] = carry[e] + plsc.all_reduce_population_count(mask)
            h, s = k // 8, k % 8
            dest_vmem[h, pl.ds(16 * s, 16)] = dk
            pairp = jnp.broadcast_to(base + 16 * k, (16,)) + lanes
            tok_vmem[h, pl.ds(16 * s, 16)] = pairp // K
        cps = []
        for h in range(2):
            c1 = pltpu.make_async_copy(tok_vmem.at[h],
                                       tok_hbm.at[dest_vmem.at[h]],
                                       sem0 if h == 0 else sem1)
            c1.start()
            c2 = pltpu.make_async_copy(g_vmem.at[h],
                                       gate_hbm.at[dest_vmem.at[h]],
                                       sem2 if h == 0 else sem3)
            c2.start()
            cps += [c1, c2]
        for c in cps:
            c.wait()

        @pl.when(wid == 0)
        def _bmap():
            total = cnb[E - 1] + nb[E - 1]
            belast = zero16
            for e in range(1, E):
                belast = belast + (total - 1 >= cnb[e]).astype(jnp.int32)
            for j in range(2):
                bvec = lanes + 16 * j
                acc = zero16
                for e in range(1, E):
                    acc = acc + (bvec >= cnb[e]).astype(jnp.int32)
                acc = jnp.where(bvec < total, acc, belast)
                be_vmem[pl.ds(16 * j, 16)] = acc
            pltpu.sync_copy(be_vmem.at[pl.ds(0, NBLK)], be_hbm)


def _sc_dispatch(topi, topg):
    mesh = plsc.VectorSubcoreMesh(core_axis_name="c", subcore_axis_name="s")
    cp = pltpu.CompilerParams()
    if "needs_layout_passes" in pltpu.CompilerParams.__dataclass_fields__:
        cp = dataclasses.replace(cp, needs_layout_passes=False)
    fn = pl.kernel(
        _dispatch_body, mesh=mesh, compiler_params=cp,
        out_type=[jax.ShapeDtypeStruct((NPAD,), jnp.int32),
                  jax.ShapeDtypeStruct((NPAD,), jnp.float32),
                  jax.ShapeDtypeStruct((NBLK,), jnp.int32)],
        scratch_types=[pltpu.VMEM((CHUNK,), jnp.int32),
                       pltpu.VMEM((2, 128), jnp.float32),
                       pltpu.VMEM((2, 128), jnp.int32),
                       pltpu.VMEM((2, 128), jnp.int32),
                       pltpu.VMEM((16,), jnp.int32),
                       pltpu.VMEM((16,), jnp.int32),
                       pltpu.VMEM((16,), jnp.int32),
                       pltpu.VMEM((ZF,), jnp.int32),
                       pltpu.VMEM((ZF,), jnp.float32),
                       pltpu.VMEM((32,), jnp.int32),
                       pltpu.VMEM((NT * 8,), jnp.int32),
                       pltpu.VMEM_SHARED((NT * 8,), jnp.int32),
                       pltpu.SemaphoreType.DMA,
                       pltpu.SemaphoreType.DMA,
                       pltpu.SemaphoreType.DMA,
                       pltpu.SemaphoreType.DMA],
    )
    tok_flat, gate_flat, block_expert = fn(topi.reshape(T * K),
                                           topg.reshape(T * K))
    return (tok_flat.reshape(NBLK, BLK, 1),
            gate_flat.reshape(NBLK, 1, BLK), block_expert)


def _ffn_kernel(be_ref, lin_ref, tok_ref, gate_ref, w1_ref, b1_ref,
                w2_ref, b2_ref, xsum_ref, y_ref, out_ref, acc_scr):
    i = pl.program_id(0)

    @pl.when(i == 0)
    def _prologue():
        acc_scr[...] = jnp.zeros_like(acc_scr)

    gates = gate_ref[0]
    has_rows = jnp.sum(gates) > 0.0

    @pl.when(has_rows)
    def _block():
        tok = tok_ref[0]
        onehot = (tok == jax.lax.broadcasted_iota(jnp.int32, (BLK, T), 1)
                  ).astype(jnp.float8_e4m3fn)
        xg = jax.lax.dot(onehot, lin_ref[...],
                         preferred_element_type=jnp.float32)
        h = jax.lax.dot(xg.astype(jnp.bfloat16),
                        w1_ref[0].astype(jnp.bfloat16),
                        preferred_element_type=jnp.float32)
        h = jax.nn.gelu(h + b1_ref[0])
        eo = jax.lax.dot(h.astype(jnp.bfloat16),
                         w2_ref[0].astype(jnp.bfloat16),
                         preferred_element_type=jnp.float32)
        contrib = jax.lax.dot(gates, eo,
                              preferred_element_type=jnp.float32)
        acc_scr[...] += contrib + jnp.sum(gates) * b2_ref[0]

    @pl.when(i == NBLK - 1)
    def _epilogue():
        sent = (xsum_ref[...] + acc_scr[...]) * (1.0 / T)
        mx = jnp.max(sent)
        lse = mx + jnp.log(jnp.sum(jnp.exp(sent - mx)))
        cls = jax.lax.broadcasted_iota(jnp.int32, (1, D), 1)
        picked = jnp.sum(jnp.where(cls == y_ref[0, 0], sent, 0.0))
        out_ref[...] = jnp.broadcast_to(lse - picked, (1, 1))


@jax.jit
def _run(x, y, W_lin, b_lin, Wg, W1, b1, W2, b2):
    x2 = x.reshape(T, D)
    y32 = y.astype(jnp.int32).reshape(1, 1)
    blin2 = b_lin.reshape(1, D)

    topi, topg = pl.pallas_call(
        _router_kernel,
        in_specs=[pl.BlockSpec((T, D), lambda: (0, 0)),
                  pl.BlockSpec((D, D), lambda: (0, 0)),
                  pl.BlockSpec((1, D), lambda: (0, 0)),
                  pl.BlockSpec((D, E), lambda: (0, 0))],
        out_specs=[pl.BlockSpec((T, K), lambda: (0, 0)),
                   pl.BlockSpec((T, K), lambda: (0, 0))],
        out_shape=[jax.ShapeDtypeStruct((T, K), jnp.int32),
                   jax.ShapeDtypeStruct((T, K), jnp.float32)],
    )(x2, W_lin, blin2, Wg)

    lin, xsum = pl.pallas_call(
        _lin_kernel,
        in_specs=[pl.BlockSpec((T, D), lambda: (0, 0)),
                  pl.BlockSpec((D, D), lambda: (0, 0)),
                  pl.BlockSpec((1, D), lambda: (0, 0))],
        out_specs=[pl.BlockSpec((T, D), lambda: (0, 0)),
                   pl.BlockSpec((1, D), lambda: (0, 0))],
        out_shape=[jax.ShapeDtypeStruct((T, D), jnp.float8_e4m3fn),
                   jax.ShapeDtypeStruct((1, D), jnp.float32)],
    )(x2, W_lin, blin2)

    tok_sched, gate_sched, block_expert = _sc_dispatch(topi, topg)

    grid_spec = pltpu.PrefetchScalarGridSpec(
        num_scalar_prefetch=1,
        grid=(NBLK,),
        in_specs=[
            pl.BlockSpec((T, D), lambda i, be: (0, 0)),
            pl.BlockSpec((1, BLK, 1), lambda i, be: (i, 0, 0)),
            pl.BlockSpec((1, 1, BLK), lambda i, be: (i, 0, 0)),
            pl.BlockSpec((1, D, FF), lambda i, be: (be[i], 0, 0)),
            pl.BlockSpec((1, 1, FF), lambda i, be: (be[i], 0, 0)),
            pl.BlockSpec((1, FF, D), lambda i, be: (be[i], 0, 0)),
            pl.BlockSpec((1, 1, D), lambda i, be: (be[i], 0, 0)),
            pl.BlockSpec((1, D), lambda i, be: (0, 0)),
            pl.BlockSpec(memory_space=pltpu.SMEM),
        ],
        out_specs=pl.BlockSpec((1, 1), lambda i, be: (0, 0)),
        scratch_shapes=[pltpu.VMEM((1, D), jnp.float32)],
    )
    out = pl.pallas_call(
        _ffn_kernel,
        grid_spec=grid_spec,
        out_shape=jax.ShapeDtypeStruct((1, 1), jnp.float32),
        compiler_params=pltpu.CompilerParams(
            dimension_semantics=("arbitrary",),
        ),
    )(block_expert, lin, tok_sched, gate_sched, W1, b1.reshape(E, 1, FF),
      W2, b2.reshape(E, 1, D), xsum, y32)
    return out[0, 0]


def kernel(x, y, W_lin, b_lin, Wg, W1, b1, W2, b2):
    return _run(x, y, W_lin, b_lin, Wg, W1, b1, W2, b2)


# speedup vs baseline: 1.3015x; 1.0356x over previous
import dataclasses
import functools

import jax
import jax.numpy as jnp
from jax import lax
from jax.experimental import pallas as pl
from jax.experimental.pallas import tpu as pltpu
from jax.experimental.pallas import tpu_sc as plsc

B, S, D = 1, 2048, 768
E, K, FF = 8, 2, 3072
T = B * S
BLK = 256
NBLK = (T * K) // BLK + E
NPAD = NBLK * BLK

NT = 16
CHUNK = (T * K) // NT
ZF = NPAD // NT


def _router_kernel(x_ref, wlin_ref, blin_ref, wg_ref, topi_ref, topg_ref):
    wc = jax.lax.dot(wlin_ref[...], wg_ref[...],
                     preferred_element_type=jnp.float32)
    logits = (jax.lax.dot(x_ref[...], wc,
                          preferred_element_type=jnp.float32)
              + jax.lax.dot(blin_ref[...], wg_ref[...],
                            preferred_element_type=jnp.float32))
    m = jnp.max(logits, axis=1, keepdims=True)
    p = jnp.exp(logits - m)
    p = p / jnp.sum(p, axis=1, keepdims=True)
    eids = jax.lax.broadcasted_iota(jnp.int32, (T, E), 1)
    v1 = jnp.max(p, axis=1, keepdims=True)
    i1 = jnp.min(jnp.where(p == v1, eids, E), axis=1, keepdims=True)
    p2 = jnp.where(eids == i1, -1.0, p)
    v2 = jnp.max(p2, axis=1, keepdims=True)
    i2 = jnp.min(jnp.where(p2 == v2, eids, E), axis=1, keepdims=True)
    den = v1 + v2
    topi_ref[...] = jnp.concatenate([i1, i2], axis=1)
    topg_ref[...] = jnp.concatenate([v1 / den, v2 / den], axis=1)


def _lin_kernel(x_ref, wlin_ref, blin_ref, lin_ref, xsum_ref):
    xb = x_ref[...]
    lin = jax.lax.dot(xb.astype(jnp.bfloat16),
                      wlin_ref[...].astype(jnp.bfloat16),
                      preferred_element_type=jnp.float32)
    lin_ref[...] = (lin + blin_ref[...]).astype(jnp.float8_e4m3fn)
    xsum_ref[...] = jnp.sum(xb, axis=0, keepdims=True)


def _dispatch_body(topi_hbm, topg_hbm, tok_hbm, gate_hbm, be_hbm,
                   e_vmem, g_vmem, tok_vmem, dest_vmem, cnt_vmem,
                   sc0_vmem, sc1_vmem, zf_i, zf_f, be_vmem,
                   counts_vmem, counts_sh, sem0, sem1, sem2, sem3):
    wid = lax.axis_index("s")
    cid = lax.axis_index("c")
    lanes = jnp.arange(16, dtype=jnp.int32)
    zero16 = jnp.zeros((16,), jnp.int32)
    on_core0 = cid == 0

    def splat(x, e):
        return plsc.load_gather(x, [jnp.full((16,), e, jnp.int32)])

    @pl.when(on_core0)
    def _phase1():
        base = wid * CHUNK
        cp_e = pltpu.make_async_copy(topi_hbm.at[pl.ds(base, CHUNK)],
                                     e_vmem, sem0)
        cp_e.start()
        cp_g0 = pltpu.make_async_copy(topg_hbm.at[pl.ds(base, 128)],
                                      g_vmem.at[0], sem1)
        cp_g0.start()
        cp_g1 = pltpu.make_async_copy(topg_hbm.at[pl.ds(base + 128, 128)],
                                      g_vmem.at[1], sem2)
        cp_g1.start()
        zi = jnp.zeros((16,), jnp.int32)
        zf32 = jnp.zeros((16,), jnp.float32)
        for j in range(ZF // 16):
            zf_i[pl.ds(16 * j, 16)] = zi
            zf_f[pl.ds(16 * j, 16)] = zf32
        cp_z0 = pltpu.make_async_copy(zf_i, tok_hbm.at[pl.ds(wid * ZF, ZF)],
                                      sem3)
        cp_z0.start()
        cp_e.wait()
        cnt = [zero16] * E
        for k in range(CHUNK // 16):
            v = e_vmem[pl.ds(16 * k, 16)]
            for e in range(E):
                cnt[e] = cnt[e] + plsc.all_reduce_population_count(v == e)
        cvec = zero16
        for e in range(E):
            cvec = jnp.where(lanes == e, cnt[e], cvec)
        cnt_vmem[...] = cvec
        pltpu.sync_copy(cnt_vmem.at[pl.ds(0, 8)],
                        counts_sh.at[pl.ds(wid * 8, 8)])
        cp_z0.wait()
        cp_z1 = pltpu.make_async_copy(zf_f, gate_hbm.at[pl.ds(wid * ZF, ZF)],
                                      sem3)
        cp_z1.start()
        cp_g0.wait()
        cp_g1.wait()
        cp_z1.wait()

    plsc.subcore_barrier()

    @pl.when(on_core0)
    def _phase2():
        base = wid * CHUNK
        pltpu.sync_copy(counts_sh, counts_vmem)
        tot16 = zero16
        pfx16 = zero16
        for k in range(NT // 2):
            ck = counts_vmem[pl.ds(16 * k, 16)]
            tot16 = tot16 + ck
            wvec = 2 * k + (lanes >= 8).astype(jnp.int32)
            pfx16 = pfx16 + jnp.where(wvec < jnp.broadcast_to(wid, (16,)),
                                      ck, zero16)
        sc0_vmem[...] = tot16
        sc1_vmem[...] = pfx16
        tot = [splat(sc0_vmem, e) + splat(sc0_vmem, e + 8) for e in range(E)]
        pfx = [splat(sc1_vmem, e) + splat(sc1_vmem, e + 8) for e in range(E)]
        nb = [(tot[e] + (BLK - 1)) // BLK for e in range(E)]
        cnb = [zero16] * E
        for e in range(1, E):
            cnb[e] = cnb[e - 1] + nb[e - 1]
        basee = [cnb[e] * BLK + pfx[e] for e in range(E)]
        carry = [zero16] * E
        for k in range(CHUNK // 16):
            v = e_vmem[pl.ds(16 * k, 16)]
            dk = zero16
            for e in range(E):
                mask = v == e
                rank = plsc.cumsum(mask.astype(jnp.int32)) - 1
                dk = jnp.where(mask, basee[e] + carry[e] + rank, dk)
                carry[e] = carry[e] + plsc.all_reduce_population_count(mask)
            h, s = k // 8, k % 8
            dest_vmem[h, pl.ds(16 * s, 16)] = dk
            pairp = jnp.broadcast_to(base + 16 * k, (16,)) + lanes
            tok_vmem[h, pl.ds(16 * s, 16)] = pairp // K
        cps = []
        for h in range(2):
            c1 = pltpu.make_async_copy(tok_vmem.at[h],
                                       tok_hbm.at[dest_vmem.at[h]],
                                       sem0 if h == 0 else sem1)
            c1.start()
            c2 = pltpu.make_async_copy(g_vmem.at[h],
                                       gate_hbm.at[dest_vmem.at[h]],
                                       sem2 if h == 0 else sem3)
            c2.start()
            cps += [c1, c2]
        for c in cps:
            c.wait()

        @pl.when(wid == 0)
        def _bmap():
            total = cnb[E - 1] + nb[E - 1]
            belast = zero16
            for e in range(1, E):
                belast = belast + (total - 1 >= cnb[e]).astype(jnp.int32)
            for j in range(2):
                bvec = lanes + 16 * j
                acc = zero16
                for e in range(1, E):
                    acc = acc + (bvec >= cnb[e]).astype(jnp.int32)
                acc = jnp.where(bvec < total, acc, belast)
                be_vmem[pl.ds(16 * j, 16)] = acc
            pltpu.sync_copy(be_vmem.at[pl.ds(0, NBLK)], be_hbm)


def _sc_dispatch(topi, topg):
    mesh = plsc.VectorSubcoreMesh(core_axis_name="c", subcore_axis_name="s")
    cp = pltpu.CompilerParams()
    if "needs_layout_passes" in pltpu.CompilerParams.__dataclass_fields__:
        cp = dataclasses.replace(cp, needs_layout_passes=False)
    fn = pl.kernel(
        _dispatch_body, mesh=mesh, compiler_params=cp,
        out_type=[jax.ShapeDtypeStruct((NPAD,), jnp.int32),
                  jax.ShapeDtypeStruct((NPAD,), jnp.float32),
                  jax.ShapeDtypeStruct((NBLK,), jnp.int32)],
        scratch_types=[pltpu.VMEM((CHUNK,), jnp.int32),
                       pltpu.VMEM((2, 128), jnp.float32),
                       pltpu.VMEM((2, 128), jnp.int32),
                       pltpu.VMEM((2, 128), jnp.int32),
                       pltpu.VMEM((16,), jnp.int32),
                       pltpu.VMEM((16,), jnp.int32),
                       pltpu.VMEM((16,), jnp.int32),
                       pltpu.VMEM((ZF,), jnp.int32),
                       pltpu.VMEM((ZF,), jnp.float32),
                       pltpu.VMEM((32,), jnp.int32),
                       pltpu.VMEM((NT * 8,), jnp.int32),
                       pltpu.VMEM_SHARED((NT * 8,), jnp.int32),
                       pltpu.SemaphoreType.DMA,
                       pltpu.SemaphoreType.DMA,
                       pltpu.SemaphoreType.DMA,
                       pltpu.SemaphoreType.DMA],
    )
    tok_flat, gate_flat, block_expert = fn(topi.reshape(T * K),
                                           topg.reshape(T * K))
    return (tok_flat.reshape(NBLK, BLK, 1),
            gate_flat.reshape(NBLK, 1, BLK), block_expert)


def _ffn_kernel(be_ref, lin_ref, tok_ref, gate_ref, w1_ref, b1_ref,
                w2_ref, b2_ref, xsum_ref, y_ref, out_ref, acc_scr):
    i = pl.program_id(0)

    @pl.when(i == 0)
    def _prologue():
        acc_scr[...] = jnp.zeros_like(acc_scr)

    gates = gate_ref[0]
    has_rows = jnp.sum(gates) > 0.0

    @pl.when(has_rows)
    def _block():
        tok = tok_ref[0]
        onehot = (tok == jax.lax.broadcasted_iota(jnp.int32, (BLK, T), 1)
                  ).astype(jnp.float8_e4m3fn)
        xg = jax.lax.dot(onehot, lin_ref[...],
                         preferred_element_type=jnp.float32)
        h = jax.lax.dot(xg.astype(jnp.float8_e4m3fn),
                        w1_ref[0].astype(jnp.float8_e4m3fn),
                        preferred_element_type=jnp.float32)
        h = jax.nn.gelu(h + b1_ref[0])
        eo = jax.lax.dot(h.astype(jnp.float8_e4m3fn),
                         w2_ref[0].astype(jnp.float8_e4m3fn),
                         preferred_element_type=jnp.float32)
        contrib = jax.lax.dot(gates, eo,
                              preferred_element_type=jnp.float32)
        acc_scr[...] += contrib + jnp.sum(gates) * b2_ref[0]

    @pl.when(i == NBLK - 1)
    def _epilogue():
        sent = (xsum_ref[...] + acc_scr[...]) * (1.0 / T)
        mx = jnp.max(sent)
        lse = mx + jnp.log(jnp.sum(jnp.exp(sent - mx)))
        cls = jax.lax.broadcasted_iota(jnp.int32, (1, D), 1)
        picked = jnp.sum(jnp.where(cls == y_ref[0, 0], sent, 0.0))
        out_ref[...] = jnp.broadcast_to(lse - picked, (1, 1))


@jax.jit
def _run(x, y, W_lin, b_lin, Wg, W1, b1, W2, b2):
    x2 = x.reshape(T, D)
    y32 = y.astype(jnp.int32).reshape(1, 1)
    blin2 = b_lin.reshape(1, D)

    topi, topg = pl.pallas_call(
        _router_kernel,
        in_specs=[pl.BlockSpec((T, D), lambda: (0, 0)),
                  pl.BlockSpec((D, D), lambda: (0, 0)),
                  pl.BlockSpec((1, D), lambda: (0, 0)),
                  pl.BlockSpec((D, E), lambda: (0, 0))],
        out_specs=[pl.BlockSpec((T, K), lambda: (0, 0)),
                   pl.BlockSpec((T, K), lambda: (0, 0))],
        out_shape=[jax.ShapeDtypeStruct((T, K), jnp.int32),
                   jax.ShapeDtypeStruct((T, K), jnp.float32)],
    )(x2, W_lin, blin2, Wg)

    lin, xsum = pl.pallas_call(
        _lin_kernel,
        in_specs=[pl.BlockSpec((T, D), lambda: (0, 0)),
                  pl.BlockSpec((D, D), lambda: (0, 0)),
                  pl.BlockSpec((1, D), lambda: (0, 0))],
        out_specs=[pl.BlockSpec((T, D), lambda: (0, 0)),
                   pl.BlockSpec((1, D), lambda: (0, 0))],
        out_shape=[jax.ShapeDtypeStruct((T, D), jnp.float8_e4m3fn),
                   jax.ShapeDtypeStruct((1, D), jnp.float32)],
    )(x2, W_lin, blin2)

    tok_sched, gate_sched, block_expert = _sc_dispatch(topi, topg)

    grid_spec = pltpu.PrefetchScalarGridSpec(
        num_scalar_prefetch=1,
        grid=(NBLK,),
        in_specs=[
            pl.BlockSpec((T, D), lambda i, be: (0, 0)),
            pl.BlockSpec((1, BLK, 1), lambda i, be: (i, 0, 0)),
            pl.BlockSpec((1, 1, BLK), lambda i, be: (i, 0, 0)),
            pl.BlockSpec((1, D, FF), lambda i, be: (be[i], 0, 0)),
            pl.BlockSpec((1, 1, FF), lambda i, be: (be[i], 0, 0)),
            pl.BlockSpec((1, FF, D), lambda i, be: (be[i], 0, 0)),
            pl.BlockSpec((1, 1, D), lambda i, be: (be[i], 0, 0)),
            pl.BlockSpec((1, D), lambda i, be: (0, 0)),
            pl.BlockSpec(memory_space=pltpu.SMEM),
        ],
        out_specs=pl.BlockSpec((1, 1), lambda i, be: (0, 0)),
        scratch_shapes=[pltpu.VMEM((1, D), jnp.float32)],
    )
    out = pl.pallas_call(
        _ffn_kernel,
        grid_spec=grid_spec,
        out_shape=jax.ShapeDtypeStruct((1, 1), jnp.float32),
        compiler_params=pltpu.CompilerParams(
            dimension_semantics=("arbitrary",),
        ),
    )(block_expert, lin, tok_sched, gate_sched, W1, b1.reshape(E, 1, FF),
      W2, b2.reshape(E, 1, D), xsum, y32)
    return out[0, 0]


def kernel(x, y, W_lin, b_lin, Wg, W1, b1, W2, b2):
    return _run(x, y, W_lin, b_lin, Wg, W1, b1, W2, b2)
